# Initial kernel scaffold; baseline (speedup 1.0000x reference)
import jax
import jax.numpy as jnp
from jax.experimental import pallas as pl


def kernel(x, gate_W, w1, w2, w3):
    raise NotImplementedError("write your pallas kernel here")


# speedup vs baseline: 1.7926x; 1.7926x over previous
import functools

import jax
import jax.numpy as jnp
from jax import lax
from jax.experimental import pallas as pl
from jax.experimental.pallas import tpu as pltpu
from jax.experimental.pallas import tpu_sc as plsc

_INTERPRET = False

DIM = 2048
HID = 5632
NE = 8
NTOK = 4096
NPAIR = NTOK * 2
TN = 512
RTILES = NTOK // TN
TM = 256
CAP = NPAIR + NE * TM
TROW = CAP // TM
TH1 = 512
NH1 = HID // TH1
QO = 512
NQ = DIM // QO
NC, NS = 2, 16
NWORK = NC * NS
CH = 16

@functools.cache
def _sc_mesh():
    return plsc.VectorSubcoreMesh(
        core_axis_name="c", subcore_axis_name="s",
        num_cores=NC, num_subcores=NS)


def _router_body(x_ref, g_ref, pos0_ref, pos1_ref, w0_ref, w1_ref,
                 eid_ref, misc_ref, s_probs, s_cnt, s_carry):
    s = pl.program_id(0)
    t = pl.program_id(1)
    xt = x_ref[...]
    g = g_ref[...]
    logits = lax.dot_general(xt, g, (((1,), (1,)), ((), ())),
                             preferred_element_type=jnp.float32)
    lane = lax.broadcasted_iota(jnp.int32, (TN, 128), 1)
    lg = jnp.where(lane < NE, logits, jnp.float32(-jnp.inf))
    m = jnp.max(lg, axis=1, keepdims=True)
    el = jnp.exp(lg - m)
    p = el / jnp.sum(el, axis=1, keepdims=True)
    p1 = jnp.max(p, axis=1, keepdims=True)
    i1 = jnp.min(jnp.where(p == p1, lane, 127), axis=1, keepdims=True)
    oh1 = lane == i1
    pm = jnp.where(oh1, jnp.float32(-1.0), p)
    p2 = jnp.max(pm, axis=1, keepdims=True)
    i2 = jnp.min(jnp.where(pm == p2, lane, 127), axis=1, keepdims=True)
    oh2 = lane == i2
    ohf = (oh1 | oh2).astype(jnp.float32)

    @pl.when((s == 0) & (t == 0))
    def _():
        s_probs[...] = jnp.zeros_like(s_probs)
        s_cnt[...] = jnp.zeros_like(s_cnt)

    @pl.when(s == 0)
    def _():
        s_probs[...] += jnp.sum(p, axis=0, keepdims=True)
        s_cnt[...] += jnp.sum(ohf, axis=0, keepdims=True)

    @pl.when(s == 1)
    def _():
        @pl.when(t == 0)
        def _():
            s_carry[...] = jnp.zeros_like(s_carry)

        ri = lax.broadcasted_iota(jnp.int32, (TN, TN), 0)
        ci = lax.broadcasted_iota(jnp.int32, (TN, TN), 1)
        ltri = (ri > ci).astype(jnp.float32)
        rank = lax.dot_general(ltri, ohf, (((1,), (0,)), ((), ())),
                               preferred_element_type=jnp.float32)
        rank = rank + s_carry[...]
        s_carry[...] += jnp.sum(ohf, axis=0, keepdims=True)

        offs = []
        aes = []
        run = jnp.float32(0.0)
        for e in range(NE):
            c = s_cnt[0, e]
            al = jnp.ceil(c / TM) * TM
            offs.append(run)
            run = run + al
            aes.append(run)
        lane1 = lane[0:1, :]
        offvec = jnp.zeros((1, 128), jnp.float32)
        for e in range(NE):
            offvec = jnp.where(lane1 == e, offs[e], offvec)
        off1 = jnp.sum(jnp.where(oh1, offvec, 0.0), axis=1, keepdims=True)
        off2 = jnp.sum(jnp.where(oh2, offvec, 0.0), axis=1, keepdims=True)
        r1 = jnp.sum(jnp.where(oh1, rank, 0.0), axis=1, keepdims=True)
        r2 = jnp.sum(jnp.where(oh2, rank, 0.0), axis=1, keepdims=True)
        pos0_ref[...] = (off1 + r1).astype(jnp.int32)
        pos1_ref[...] = (off2 + r2).astype(jnp.int32)
        wsum = p1 + p2
        w0_ref[...] = p1 / wsum
        w1_ref[...] = p2 / wsum

        @pl.when(t == RTILES - 1)
        def _():
            ps = s_probs[...]
            cnt = s_cnt[...]
            aux = jnp.sum(ps * cnt) * jnp.float32(NE / (NTOK * NTOK))
            misc_ref[...] = jnp.zeros((8, 128), jnp.float32)
            misc_ref[0:1, :] = ps
            misc_ref[1:2, :] = cnt
            misc_ref[2:3, :] = jnp.full((1, 128), aux, jnp.float32)
            tstart = (lane1 * TM).astype(jnp.float32)
            eidv = jnp.zeros((1, 128), jnp.float32)
            for e in range(NE):
                eidv += jnp.where(tstart >= aes[e], 1.0, 0.0)
            eid_ref[...] = jnp.minimum(eidv, NE - 1).astype(jnp.int32)


def _router(xf, gate_pad):
    return pl.pallas_call(
        _router_body,
        grid=(2, RTILES),
        in_specs=[
            pl.BlockSpec((TN, DIM), lambda s, t: (t, 0)),
            pl.BlockSpec((128, DIM), lambda s, t: (0, 0)),
        ],
        out_specs=[
            pl.BlockSpec((TN, 1), lambda s, t: (t, 0)),
            pl.BlockSpec((TN, 1), lambda s, t: (t, 0)),
            pl.BlockSpec((TN, 1), lambda s, t: (t, 0)),
            pl.BlockSpec((TN, 1), lambda s, t: (t, 0)),
            pl.BlockSpec((1, 128), lambda s, t: (0, 0)),
            pl.BlockSpec((8, 128), lambda s, t: (0, 0)),
        ],
        out_shape=[
            jax.ShapeDtypeStruct((NTOK, 1), jnp.int32),
            jax.ShapeDtypeStruct((NTOK, 1), jnp.int32),
            jax.ShapeDtypeStruct((NTOK, 1), jnp.float32),
            jax.ShapeDtypeStruct((NTOK, 1), jnp.float32),
            jax.ShapeDtypeStruct((1, 128), jnp.int32),
            jax.ShapeDtypeStruct((8, 128), jnp.float32),
        ],
        scratch_shapes=[
            pltpu.VMEM((1, 128), jnp.float32),
            pltpu.VMEM((1, 128), jnp.float32),
            pltpu.VMEM((1, 128), jnp.float32),
        ],
        interpret=_INTERPRET,
    )(xf, gate_pad)


def _meta_body(pos0_h, pos1_h, w0_h, w1_h, ts_h, ws_h,
               ts_v, ws_v, pidx_v, pval_v):
    cid = lax.axis_index("c")
    sid = lax.axis_index("s")

    @pl.when((cid == 0) & (sid == 0))
    def _():
        def zero16(i, carry):
            ts_v[pl.ds(i * 16, 16)] = jnp.zeros((16,), jnp.int32)
            ws_v[pl.ds(i * 16, 16)] = jnp.zeros((16,), jnp.float32)
            return carry
        lax.fori_loop(0, CAP // 16, zero16, 0)

        def do_slot(ph, vh):
            pltpu.sync_copy(ph, pidx_v)
            pltpu.sync_copy(vh, pval_v)

            def scat(i, carry):
                base = i * 16
                pv = pidx_v[pl.ds(base, 16)]
                toks = lax.iota(jnp.int32, 16) + base
                plsc.store_scatter(ts_v, [pv], toks)
                vv = pval_v[pl.ds(base, 16)]
                plsc.store_scatter(ws_v, [pv], vv)
                return carry
            lax.fori_loop(0, NTOK // 16, scat, 0)

        do_slot(pos0_h, w0_h)
        do_slot(pos1_h, w1_h)
        pltpu.sync_copy(ts_v, ts_h)
        pltpu.sync_copy(ws_v, ws_h)


def _meta(pos0, pos1, w0, w1):
    return pl.kernel(
        _meta_body,
        out_type=(jax.ShapeDtypeStruct((CAP,), jnp.int32),
                  jax.ShapeDtypeStruct((CAP,), jnp.float32)),
        mesh=_sc_mesh(),
        scratch_types=(
            pltpu.VMEM((CAP,), jnp.int32),
            pltpu.VMEM((CAP,), jnp.float32),
            pltpu.VMEM((NTOK,), jnp.int32),
            pltpu.VMEM((NTOK,), jnp.float32),
        ),
        compiler_params=pltpu.CompilerParams(needs_layout_passes=False),
        interpret=_INTERPRET,
    )(pos0, pos1, w0, w1)


def _disp_body(ts_h, x_h, xs_h, idx_v, rows_v, sem):
    cid = lax.axis_index("c")
    sid = lax.axis_index("s")
    w = sid * NC + cid
    per = CAP // NWORK
    base = w * per

    def it(i, carry):
        b = base + i * CH
        pltpu.sync_copy(ts_h.at[pl.ds(b, CH)], idx_v)
        pltpu.async_copy(x_h.at[idx_v], rows_v, sem).wait()
        pltpu.sync_copy(rows_v, xs_h.at[pl.ds(b, CH)])
        return carry
    lax.fori_loop(0, per // CH, it, 0)


def _dispatch(token_src, xf):
    return pl.kernel(
        _disp_body,
        out_type=jax.ShapeDtypeStruct((CAP, DIM), jnp.float32),
        mesh=_sc_mesh(),
        scratch_types=(
            pltpu.VMEM((CH,), jnp.int32),
            pltpu.VMEM((CH, DIM), jnp.float32),
            pltpu.SemaphoreType.DMA,
        ),
        interpret=_INTERPRET,
    )(token_src, xf)


def _gath_body(p0_h, p1_h, y_h, y0_h, y1_h, idx_v, rows_v, sem):
    cid = lax.axis_index("c")
    sid = lax.axis_index("s")
    w = sid * NC + cid
    per = NTOK // NWORK
    base = w * per

    def it(i, carry):
        b = base + i * CH
        pltpu.sync_copy(p0_h.at[pl.ds(b, CH)], idx_v)
        pltpu.async_copy(y_h.at[idx_v], rows_v, sem).wait()
        pltpu.sync_copy(rows_v, y0_h.at[pl.ds(b, CH)])
        pltpu.sync_copy(p1_h.at[pl.ds(b, CH)], idx_v)
        pltpu.async_copy(y_h.at[idx_v], rows_v, sem).wait()
        pltpu.sync_copy(rows_v, y1_h.at[pl.ds(b, CH)])
        return carry
    lax.fori_loop(0, per // CH, it, 0)


def _gather2(pos0f, pos1f, y):
    return pl.kernel(
        _gath_body,
        out_type=(jax.ShapeDtypeStruct((NTOK, DIM), jnp.float32),
                  jax.ShapeDtypeStruct((NTOK, DIM), jnp.float32)),
        mesh=_sc_mesh(),
        scratch_types=(
            pltpu.VMEM((CH,), jnp.int32),
            pltpu.VMEM((CH, DIM), jnp.float32),
            pltpu.SemaphoreType.DMA,
        ),
        interpret=_INTERPRET,
    )(pos0f, pos1f, y)


def _ffn1_body(eid_ref, xs_ref, w1_ref, w3_ref, h_ref):
    xt = xs_ref[...]
    a = lax.dot_general(xt, w1_ref[0], (((1,), (1,)), ((), ())),
                        preferred_element_type=jnp.float32)
    b = lax.dot_general(xt, w3_ref[0], (((1,), (1,)), ((), ())),
                        preferred_element_type=jnp.float32)
    h_ref[...] = (a / (1.0 + jnp.exp(-a))) * b


def _ffn1(eid, xs, w1, w3):
    return pl.pallas_call(
        _ffn1_body,
        grid_spec=pltpu.PrefetchScalarGridSpec(
            num_scalar_prefetch=1,
            grid=(NH1, TROW),
            in_specs=[
                pl.BlockSpec((TM, DIM), lambda j, t, eid: (t, 0)),
                pl.BlockSpec((1, TH1, DIM), lambda j, t, eid: (eid[t], j, 0)),
                pl.BlockSpec((1, TH1, DIM), lambda j, t, eid: (eid[t], j, 0)),
            ],
            out_specs=pl.BlockSpec((TM, TH1), lambda j, t, eid: (t, j)),
        ),
        out_shape=jax.ShapeDtypeStruct((CAP, HID), jnp.float32),
        interpret=_INTERPRET,
    )(eid, xs, w1, w3)


def _ffn2_body(eid_ref, h_ref, w2_ref, ws_ref, y_ref):
    y = lax.dot_general(h_ref[...], w2_ref[0], (((1,), (1,)), ((), ())),
                        preferred_element_type=jnp.float32)
    y_ref[...] = y * ws_ref[...]


def _ffn2(eid, h, w2, ws):
    return pl.pallas_call(
        _ffn2_body,
        grid_spec=pltpu.PrefetchScalarGridSpec(
            num_scalar_prefetch=1,
            grid=(NQ, TROW),
            in_specs=[
                pl.BlockSpec((TM, HID), lambda q, t, eid: (t, 0)),
                pl.BlockSpec((1, QO, HID), lambda q, t, eid: (eid[t], q, 0)),
                pl.BlockSpec((TM, 1), lambda q, t, eid: (t, 0)),
            ],
            out_specs=pl.BlockSpec((TM, QO), lambda q, t, eid: (t, q)),
        ),
        out_shape=jax.ShapeDtypeStruct((CAP, DIM), jnp.float32),
        interpret=_INTERPRET,
    )(eid, h, w2, ws)


def _add_body(a_ref, b_ref, o_ref):
    o_ref[...] = a_ref[...] + b_ref[...]


def _add(y0, y1):
    return pl.pallas_call(
        _add_body,
        grid=(RTILES,),
        in_specs=[
            pl.BlockSpec((TN, DIM), lambda t: (t, 0)),
            pl.BlockSpec((TN, DIM), lambda t: (t, 0)),
        ],
        out_specs=pl.BlockSpec((TN, DIM), lambda t: (t, 0)),
        out_shape=jax.ShapeDtypeStruct((NTOK, DIM), jnp.float32),
        interpret=_INTERPRET,
    )(y0, y1)


def kernel(x, gate_W, w1, w2, w3):
    B, T, C = x.shape
    xf = x.reshape(-1, C)
    gate_pad = jnp.zeros((128, DIM), jnp.float32).at[:NE].set(gate_W)

    pos0, pos1, w0, w1r, eidv, misc = _router(xf, gate_pad)
    eid = eidv[0, :TROW]
    p0f = pos0.reshape(NTOK)
    p1f = pos1.reshape(NTOK)

    token_src, ws = _meta(p0f, p1f, w0.reshape(NTOK), w1r.reshape(NTOK))
    xs = _dispatch(token_src, xf)
    h = _ffn1(eid, xs, w1, w3)
    y = _ffn2(eid, h, w2, ws.reshape(CAP, 1))
    y0, y1 = _gather2(p0f, p1f, y)
    out = _add(y0, y1)

    aux_loss = misc[2, 0]
    return out.reshape(B, T, C), aux_loss
